# baseline (speedup 1.0000x reference)
import functools

import jax
import jax.numpy as jnp
from jax import lax
from jax.experimental import pallas as pl
from jax.experimental.pallas import tpu as pltpu
from jax.experimental.pallas import tpu_sc as plsc

_E = 8
_K = 2
_T = 2048
_D = 768
_F = 2048
_TILE = 128
_P = _T * _K
_MT = _P // _TILE + _E
_S = _MT * _TILE

_NC = 2
_NS = 16
_NW = _NC * _NS
_SPW = _S // _NW
_DCH = 2
_DCL = _SPW // _DCH
_TPW = _T // _NW

def _worker_id():
    return lax.axis_index("s") * _NC + lax.axis_index("c")


@functools.lru_cache(maxsize=None)
def _sc_kernels():
    mesh = plsc.VectorSubcoreMesh(
        core_axis_name="c", subcore_axis_name="s",
        num_cores=_NC, num_subcores=_NS)

    @functools.partial(
        pl.kernel,
        out_type=jax.ShapeDtypeStruct((_S, _D), jnp.float32),
        mesh=mesh,
        scratch_types=[
            pltpu.VMEM((_DCL,), jnp.int32),
            pltpu.VMEM((_DCL,), jnp.int32),
            pltpu.VMEM((_DCL, _D), jnp.float32),
            pltpu.SemaphoreType.DMA,
        ],
    )
    def sc_dispatch(x_hbm, rows_hbm, xs_hbm, idx0, idx1, buf, sem):
        wid = _worker_id()
        base = wid * _SPW
        pltpu.sync_copy(rows_hbm.at[wid, 0], idx0)
        pltpu.sync_copy(rows_hbm.at[wid, 1], idx1)
        pltpu.async_copy(x_hbm.at[idx0], buf, sem).wait()
        pltpu.sync_copy(buf, xs_hbm.at[pl.ds(base, _DCL)])
        pltpu.async_copy(x_hbm.at[idx1], buf, sem).wait()
        pltpu.sync_copy(buf, xs_hbm.at[pl.ds(base + _DCL, _DCL)])

    @functools.partial(
        pl.kernel,
        out_type=jax.ShapeDtypeStruct((_T, _D), jnp.float32),
        mesh=mesh,
        scratch_types=[
            pltpu.VMEM((_TPW,), jnp.int32),
            pltpu.VMEM((_TPW,), jnp.int32),
            pltpu.VMEM((_TPW, _D), jnp.float32),
            pltpu.VMEM((_TPW, _D), jnp.float32),
            pltpu.SemaphoreType.DMA,
            pltpu.SemaphoreType.DMA,
        ],
    )
    def sc_combine(y_hbm, inv_hbm, out_hbm, idx0, idx1, buf0, buf1, sem0, sem1):
        wid = _worker_id()
        base = wid * _TPW
        pltpu.sync_copy(inv_hbm.at[0, wid], idx0)
        pltpu.sync_copy(inv_hbm.at[1, wid], idx1)
        cp0 = pltpu.async_copy(y_hbm.at[idx0], buf0, sem0)
        cp1 = pltpu.async_copy(y_hbm.at[idx1], buf1, sem1)
        cp0.wait()
        cp1.wait()

        def add_row(r, carry):
            for c in range(_D // 16):
                sl = pl.ds(c * 16, 16)
                buf0[r, sl] = buf0[r, sl] + buf1[r, sl]
            return carry

        lax.fori_loop(0, _TPW, add_row, 0)
        pltpu.sync_copy(buf0, out_hbm.at[pl.ds(base, _TPW)])

    return sc_dispatch, sc_combine


def _ffn_body(te_ref, xs_ref, g_ref, w1_ref, b1_ref, w2_ref, b2_ref, y_ref):
    del te_ref
    xg = xs_ref[...]
    h = jnp.dot(xg, w1_ref[0], preferred_element_type=jnp.float32)
    h = jax.nn.gelu(h + b1_ref[0])
    y = jnp.dot(h, w2_ref[0], preferred_element_type=jnp.float32)
    y_ref[...] = (y + b2_ref[0]) * g_ref[...]


def _ffn_grid_spec():
    return pltpu.PrefetchScalarGridSpec(
        num_scalar_prefetch=1,
        grid=(_MT,),
        in_specs=[
            pl.BlockSpec((_TILE, _D), lambda i, te: (i, 0)),
            pl.BlockSpec((_TILE, 1), lambda i, te: (i, 0)),
            pl.BlockSpec((1, _D, _F), lambda i, te: (te[i], 0, 0)),
            pl.BlockSpec((1, 1, _F), lambda i, te: (te[i], 0, 0)),
            pl.BlockSpec((1, _F, _D), lambda i, te: (te[i], 0, 0)),
            pl.BlockSpec((1, 1, _D), lambda i, te: (te[i], 0, 0)),
        ],
        out_specs=pl.BlockSpec((_TILE, _D), lambda i, te: (i, 0)),
    )


def _ffn(tile_e, xs, gate2d, w1, b1, w2, b2):
    return pl.pallas_call(
        _ffn_body,
        grid_spec=_ffn_grid_spec(),
        out_shape=jax.ShapeDtypeStruct((_S, _D), jnp.float32),
        compiler_params=pltpu.CompilerParams(
            dimension_semantics=("arbitrary",),
        ),
    )(tile_e, xs, gate2d, w1, b1[:, None, :], w2, b2[:, None, :])


def _route(x, router_w):
    logits = x @ router_w
    topv, topi = lax.top_k(logits, _K)
    gates = jax.nn.softmax(topv, axis=-1)
    eflat = topi.reshape(-1).astype(jnp.int32)
    gflat = gates.reshape(-1)

    onehot = (eflat[:, None] == jnp.arange(_E, dtype=jnp.int32)[None, :])
    csum = jnp.cumsum(onehot.astype(jnp.int32), axis=0)
    counts = csum[-1]
    rank = jnp.take_along_axis(csum, eflat[:, None], axis=1)[:, 0] - 1
    tiles_e = (counts + _TILE - 1) // _TILE
    tile_start = jnp.concatenate(
        [jnp.zeros(1, jnp.int32), jnp.cumsum(tiles_e).astype(jnp.int32)])
    pad_off = tile_start * _TILE
    slot = pad_off[eflat] + rank

    tok = jnp.arange(_P, dtype=jnp.int32) // _K
    row_of_slot = jnp.zeros(_S, jnp.int32).at[slot].set(tok)
    gate_of_slot = jnp.zeros(_S, jnp.float32).at[slot].set(gflat)

    tile_e = jnp.minimum(
        jnp.sum(jnp.arange(_MT, dtype=jnp.int32)[:, None] >= tile_start[None, 1:],
                axis=1),
        _E - 1).astype(jnp.int32)

    inv = slot.reshape(_T, _K)
    inv_arr = jnp.stack([inv[:, 0].reshape(_NW, _TPW),
                         inv[:, 1].reshape(_NW, _TPW)])
    rows_arr = row_of_slot.reshape(_NW, _DCH, _DCL)
    return rows_arr, gate_of_slot, tile_e, inv_arr


def kernel(x, router_w, w1, b1, w2, b2):
    rows_arr, gate_of_slot, tile_e, inv_arr = _route(x, router_w)
    sc_dispatch, sc_combine = _sc_kernels()
    xs = sc_dispatch(x, rows_arr)
    y_sorted = _ffn(tile_e, xs, gate_of_slot[:, None], w1, b1, w2, b2)
    return sc_combine(y_sorted, inv_arr)


# speedup vs baseline: 1.4158x; 1.4158x over previous
import functools

import jax
import jax.numpy as jnp
from jax import lax
from jax.experimental import pallas as pl
from jax.experimental.pallas import tpu as pltpu
from jax.experimental.pallas import tpu_sc as plsc

_E = 8
_K = 2
_T = 2048
_D = 768
_F = 2048
_TILE = 128
_P = _T * _K
_MT = _P // _TILE + _E
_S = _MT * _TILE

_NC = 2
_NS = 16
_NW = _NC * _NS
_PPW = _P // _NW
_PCH = _PPW // 2
_TPW = _T // _NW


def _worker_id():
    return lax.axis_index("s") * _NC + lax.axis_index("c")


@functools.lru_cache(maxsize=None)
def _sc_kernels():
    mesh = plsc.VectorSubcoreMesh(
        core_axis_name="c", subcore_axis_name="s",
        num_cores=_NC, num_subcores=_NS)

    @functools.partial(
        pl.kernel,
        out_type=jax.ShapeDtypeStruct((_S, _D), jnp.float32),
        mesh=mesh,
        scratch_types=[
            pltpu.VMEM((_PCH,), jnp.int32),
            pltpu.VMEM((_PCH,), jnp.int32),
            pltpu.VMEM((_PCH,), jnp.int32),
            pltpu.VMEM((_PCH,), jnp.int32),
            pltpu.VMEM((_PCH, _D), jnp.float32),
            pltpu.VMEM((_PCH, _D), jnp.float32),
            pltpu.SemaphoreType.DMA,
            pltpu.SemaphoreType.DMA,
            pltpu.SemaphoreType.DMA,
            pltpu.SemaphoreType.DMA,
        ],
    )
    def sc_dispatch(x_hbm, slots_hbm, xs_hbm,
                    sidx_a, sidx_b, tok_a, tok_b, buf_a, buf_b,
                    sem0, sem1, sem2, sem3):
        wid = _worker_id()
        tok0 = wid * (_PPW // _K)
        pltpu.sync_copy(slots_hbm.at[wid, 0], sidx_a)
        pltpu.sync_copy(slots_hbm.at[wid, 1], sidx_b)
        for k in range(_PCH // 16):
            i = lax.iota(jnp.int32, 16) + (k * 16)
            tok_a[pl.ds(k * 16, 16)] = tok0 + lax.shift_right_logical(i, 1)
            tok_b[pl.ds(k * 16, 16)] = (
                tok0 + lax.shift_right_logical(_PCH + i, 1))
        cp_a = pltpu.async_copy(x_hbm.at[tok_a], buf_a, sem0)
        cp_b = pltpu.async_copy(x_hbm.at[tok_b], buf_b, sem1)
        cp_a.wait()
        st_a = pltpu.async_copy(buf_a, xs_hbm.at[sidx_a], sem2)
        cp_b.wait()
        st_b = pltpu.async_copy(buf_b, xs_hbm.at[sidx_b], sem3)
        st_a.wait()
        st_b.wait()

    @functools.partial(
        pl.kernel,
        out_type=jax.ShapeDtypeStruct((_T, _D), jnp.float32),
        mesh=mesh,
        scratch_types=[
            pltpu.VMEM((_TPW,), jnp.int32),
            pltpu.VMEM((_TPW,), jnp.int32),
            pltpu.VMEM((_TPW, 16), jnp.float32),
            pltpu.VMEM((_TPW, 16), jnp.float32),
            pltpu.VMEM((_TPW, _D), jnp.float32),
            pltpu.VMEM((_TPW, _D), jnp.float32),
            pltpu.SemaphoreType.DMA,
            pltpu.SemaphoreType.DMA,
        ],
    )
    def sc_combine(y_hbm, inv_hbm, g_hbm, out_hbm,
                   idx0, idx1, g0, g1, buf0, buf1, sem0, sem1):
        wid = _worker_id()
        base = wid * _TPW
        pltpu.sync_copy(inv_hbm.at[0, wid], idx0)
        pltpu.sync_copy(inv_hbm.at[1, wid], idx1)
        pltpu.sync_copy(g_hbm.at[0, wid], g0)
        pltpu.sync_copy(g_hbm.at[1, wid], g1)
        cp0 = pltpu.async_copy(y_hbm.at[idx0], buf0, sem0)
        cp1 = pltpu.async_copy(y_hbm.at[idx1], buf1, sem1)
        cp0.wait()
        cp1.wait()

        def row_fn(r, carry):
            g0v = g0[r, :]
            g1v = g1[r, :]
            for c in range(_D // 16):
                sl = pl.ds(c * 16, 16)
                buf0[r, sl] = g0v * buf0[r, sl] + g1v * buf1[r, sl]
            return carry

        lax.fori_loop(0, _TPW, row_fn, 0)
        pltpu.sync_copy(buf0, out_hbm.at[pl.ds(base, _TPW)])

    return sc_dispatch, sc_combine


def _ffn_body(te_ref, xs_ref, w1_ref, b1_ref, w2_ref, b2_ref, y_ref):
    del te_ref
    xg = xs_ref[...]
    h = jnp.dot(xg, w1_ref[0], preferred_element_type=jnp.float32)
    h = jax.nn.gelu(h + b1_ref[0])
    y = jnp.dot(h, w2_ref[0], preferred_element_type=jnp.float32)
    y_ref[...] = y + b2_ref[0]


def _ffn_grid_spec():
    return pltpu.PrefetchScalarGridSpec(
        num_scalar_prefetch=1,
        grid=(_MT,),
        in_specs=[
            pl.BlockSpec((_TILE, _D), lambda i, te: (i, 0)),
            pl.BlockSpec((1, _D, _F), lambda i, te: (te[i], 0, 0)),
            pl.BlockSpec((1, 1, _F), lambda i, te: (te[i], 0, 0)),
            pl.BlockSpec((1, _F, _D), lambda i, te: (te[i], 0, 0)),
            pl.BlockSpec((1, 1, _D), lambda i, te: (te[i], 0, 0)),
        ],
        out_specs=pl.BlockSpec((_TILE, _D), lambda i, te: (i, 0)),
    )


def _ffn(tile_e, xs, w1, b1, w2, b2):
    return pl.pallas_call(
        _ffn_body,
        grid_spec=_ffn_grid_spec(),
        out_shape=jax.ShapeDtypeStruct((_S, _D), jnp.float32),
        compiler_params=pltpu.CompilerParams(
            dimension_semantics=("arbitrary",),
        ),
    )(tile_e, xs, w1, b1[:, None, :], w2, b2[:, None, :])


def _route(x, router_w):
    logits = x @ router_w
    topv, topi = lax.top_k(logits, _K)
    gates = jax.nn.softmax(topv, axis=-1)
    eflat = topi.reshape(-1).astype(jnp.int32)

    onehot = (eflat[:, None] == jnp.arange(_E, dtype=jnp.int32)[None, :])
    csum = jnp.cumsum(onehot.astype(jnp.int32), axis=0)
    counts = csum[-1]
    rank = jnp.take_along_axis(csum, eflat[:, None], axis=1)[:, 0] - 1
    tiles_e = (counts + _TILE - 1) // _TILE
    tile_start = jnp.concatenate(
        [jnp.zeros(1, jnp.int32), jnp.cumsum(tiles_e).astype(jnp.int32)])
    pad_off = tile_start * _TILE
    slot = pad_off[eflat] + rank

    tile_e = jnp.minimum(
        jnp.sum(jnp.arange(_MT, dtype=jnp.int32)[:, None] >= tile_start[None, 1:],
                axis=1),
        _E - 1).astype(jnp.int32)

    slots_arr = slot.reshape(_NW, _K, _PCH)
    inv = slot.reshape(_T, _K)
    inv_arr = jnp.stack([inv[:, 0].reshape(_NW, _TPW),
                         inv[:, 1].reshape(_NW, _TPW)])
    g_arr = jnp.broadcast_to(
        jnp.stack([gates[:, 0].reshape(_NW, _TPW),
                   gates[:, 1].reshape(_NW, _TPW)])[..., None],
        (2, _NW, _TPW, 16)).astype(jnp.float32)
    return slots_arr, tile_e, inv_arr, g_arr


def kernel(x, router_w, w1, b1, w2, b2):
    slots_arr, tile_e, inv_arr, g_arr = _route(x, router_w)
    sc_dispatch, sc_combine = _sc_kernels()
    xs = sc_dispatch(x, slots_arr)
    y_sorted = _ffn(tile_e, xs, w1, b1, w2, b2)
    return sc_combine(y_sorted, inv_arr, g_arr)


# speedup vs baseline: 1.5250x; 1.0771x over previous
import functools

import jax
import jax.numpy as jnp
from jax import lax
from jax.experimental import pallas as pl
from jax.experimental.pallas import tpu as pltpu
from jax.experimental.pallas import tpu_sc as plsc

_E = 8
_K = 2
_T = 2048
_D = 768
_F = 2048
_TILE = 256
_P = _T * _K
_MT = _P // _TILE + _E
_S = _MT * _TILE

_NC = 2
_NS = 16
_NW = _NC * _NS
_PPW = _P // _NW
_PCH = _PPW // 2
_TPW = _T // _NW


def _worker_id():
    return lax.axis_index("s") * _NC + lax.axis_index("c")


@functools.lru_cache(maxsize=None)
def _sc_kernels():
    mesh = plsc.VectorSubcoreMesh(
        core_axis_name="c", subcore_axis_name="s",
        num_cores=_NC, num_subcores=_NS)

    @functools.partial(
        pl.kernel,
        out_type=jax.ShapeDtypeStruct((_S, _D), jnp.float32),
        mesh=mesh,
        scratch_types=[
            pltpu.VMEM((_PCH,), jnp.int32),
            pltpu.VMEM((_PCH,), jnp.int32),
            pltpu.VMEM((_PCH,), jnp.int32),
            pltpu.VMEM((_PCH,), jnp.int32),
            pltpu.VMEM((_PCH, _D), jnp.float32),
            pltpu.VMEM((_PCH, _D), jnp.float32),
            pltpu.SemaphoreType.DMA,
            pltpu.SemaphoreType.DMA,
            pltpu.SemaphoreType.DMA,
            pltpu.SemaphoreType.DMA,
        ],
    )
    def sc_dispatch(x_hbm, slots_hbm, xs_hbm,
                    sidx_a, sidx_b, tok_a, tok_b, buf_a, buf_b,
                    sem0, sem1, sem2, sem3):
        wid = _worker_id()
        tok0 = wid * (_PPW // _K)
        pltpu.sync_copy(slots_hbm.at[wid, 0], sidx_a)
        pltpu.sync_copy(slots_hbm.at[wid, 1], sidx_b)
        for k in range(_PCH // 16):
            i = lax.iota(jnp.int32, 16) + (k * 16)
            tok_a[pl.ds(k * 16, 16)] = tok0 + lax.shift_right_logical(i, 1)
            tok_b[pl.ds(k * 16, 16)] = (
                tok0 + lax.shift_right_logical(_PCH + i, 1))
        cp_a = pltpu.async_copy(x_hbm.at[tok_a], buf_a, sem0)
        cp_b = pltpu.async_copy(x_hbm.at[tok_b], buf_b, sem1)
        cp_a.wait()
        st_a = pltpu.async_copy(buf_a, xs_hbm.at[sidx_a], sem2)
        cp_b.wait()
        st_b = pltpu.async_copy(buf_b, xs_hbm.at[sidx_b], sem3)
        st_a.wait()
        st_b.wait()

    @functools.partial(
        pl.kernel,
        out_type=jax.ShapeDtypeStruct((_T, _D), jnp.float32),
        mesh=mesh,
        scratch_types=[
            pltpu.VMEM((_TPW,), jnp.int32),
            pltpu.VMEM((_TPW,), jnp.int32),
            pltpu.VMEM((_TPW, 16), jnp.float32),
            pltpu.VMEM((_TPW, 16), jnp.float32),
            pltpu.VMEM((_TPW, _D), jnp.float32),
            pltpu.VMEM((_TPW, _D), jnp.float32),
            pltpu.SemaphoreType.DMA,
            pltpu.SemaphoreType.DMA,
        ],
    )
    def sc_combine(y_hbm, inv_hbm, g_hbm, out_hbm,
                   idx0, idx1, g0, g1, buf0, buf1, sem0, sem1):
        wid = _worker_id()
        base = wid * _TPW
        pltpu.sync_copy(inv_hbm.at[0, wid], idx0)
        pltpu.sync_copy(inv_hbm.at[1, wid], idx1)
        pltpu.sync_copy(g_hbm.at[0, wid], g0)
        pltpu.sync_copy(g_hbm.at[1, wid], g1)
        cp0 = pltpu.async_copy(y_hbm.at[idx0], buf0, sem0)
        cp1 = pltpu.async_copy(y_hbm.at[idx1], buf1, sem1)
        cp0.wait()
        cp1.wait()

        def row_fn(r, carry):
            g0v = g0[r, :]
            g1v = g1[r, :]
            for c in range(_D // 16):
                sl = pl.ds(c * 16, 16)
                buf0[r, sl] = g0v * buf0[r, sl] + g1v * buf1[r, sl]
            return carry

        lax.fori_loop(0, _TPW, row_fn, 0)
        pltpu.sync_copy(buf0, out_hbm.at[pl.ds(base, _TPW)])

    return sc_dispatch, sc_combine


def _ffn_body(te_ref, xs_ref, w1_ref, b1_ref, w2_ref, b2_ref, y_ref):
    del te_ref
    xg = xs_ref[...]
    h = jnp.dot(xg, w1_ref[0], preferred_element_type=jnp.float32)
    h = jax.nn.gelu(h + b1_ref[0])
    y = jnp.dot(h, w2_ref[0], preferred_element_type=jnp.float32)
    y_ref[...] = y + b2_ref[0]


def _ffn_grid_spec():
    return pltpu.PrefetchScalarGridSpec(
        num_scalar_prefetch=1,
        grid=(_MT,),
        in_specs=[
            pl.BlockSpec((_TILE, _D), lambda i, te: (i, 0)),
            pl.BlockSpec((1, _D, _F), lambda i, te: (te[i], 0, 0)),
            pl.BlockSpec((1, 1, _F), lambda i, te: (te[i], 0, 0)),
            pl.BlockSpec((1, _F, _D), lambda i, te: (te[i], 0, 0)),
            pl.BlockSpec((1, 1, _D), lambda i, te: (te[i], 0, 0)),
        ],
        out_specs=pl.BlockSpec((_TILE, _D), lambda i, te: (i, 0)),
    )


def _ffn(tile_e, xs, w1, b1, w2, b2):
    return pl.pallas_call(
        _ffn_body,
        grid_spec=_ffn_grid_spec(),
        out_shape=jax.ShapeDtypeStruct((_S, _D), jnp.float32),
        compiler_params=pltpu.CompilerParams(
            dimension_semantics=("arbitrary",),
        ),
    )(tile_e, xs, w1, b1[:, None, :], w2, b2[:, None, :])


def _route(x, router_w):
    logits = x @ router_w
    ar = jnp.arange(_E, dtype=jnp.int32)
    i0 = jnp.argmax(logits, axis=-1).astype(jnp.int32)
    v0 = jnp.max(logits, axis=-1)
    masked = jnp.where(ar[None, :] == i0[:, None], -jnp.inf, logits)
    i1 = jnp.argmax(masked, axis=-1).astype(jnp.int32)
    v1 = jnp.max(masked, axis=-1)
    topi = jnp.stack([i0, i1], axis=-1)
    topv = jnp.stack([v0, v1], axis=-1)
    gates = jax.nn.softmax(topv, axis=-1)
    eflat = topi.reshape(-1).astype(jnp.int32)

    onehot = (eflat[:, None] == jnp.arange(_E, dtype=jnp.int32)[None, :])
    csum = jnp.cumsum(onehot.astype(jnp.int32), axis=0)
    counts = csum[-1]
    rank = jnp.take_along_axis(csum, eflat[:, None], axis=1)[:, 0] - 1
    tiles_e = (counts + _TILE - 1) // _TILE
    tile_start = jnp.concatenate(
        [jnp.zeros(1, jnp.int32), jnp.cumsum(tiles_e).astype(jnp.int32)])
    pad_off = tile_start * _TILE
    slot = pad_off[eflat] + rank

    tile_e = jnp.minimum(
        jnp.sum(jnp.arange(_MT, dtype=jnp.int32)[:, None] >= tile_start[None, 1:],
                axis=1),
        _E - 1).astype(jnp.int32)

    slots_arr = slot.reshape(_NW, _K, _PCH)
    inv = slot.reshape(_T, _K)
    inv_arr = jnp.stack([inv[:, 0].reshape(_NW, _TPW),
                         inv[:, 1].reshape(_NW, _TPW)])
    g_arr = jnp.broadcast_to(
        jnp.stack([gates[:, 0].reshape(_NW, _TPW),
                   gates[:, 1].reshape(_NW, _TPW)])[..., None],
        (2, _NW, _TPW, 16)).astype(jnp.float32)
    return slots_arr, tile_e, inv_arr, g_arr


def kernel(x, router_w, w1, b1, w2, b2):
    slots_arr, tile_e, inv_arr, g_arr = _route(x, router_w)
    sc_dispatch, sc_combine = _sc_kernels()
    xs = sc_dispatch(x, slots_arr)
    y_sorted = _ffn(tile_e, xs, w1, b1, w2, b2)
    return sc_combine(y_sorted, inv_arr, g_arr)


# speedup vs baseline: 1.6384x; 1.0743x over previous
import functools

import jax
import jax.numpy as jnp
from jax import lax
from jax.experimental import pallas as pl
from jax.experimental.pallas import tpu as pltpu
from jax.experimental.pallas import tpu_sc as plsc

_E = 8
_K = 2
_T = 2048
_D = 768
_F = 2048
_TILE = 256
_P = _T * _K
_MT = _P // _TILE + _E
_S = _MT * _TILE

_NC = 2
_NS = 16
_NW = _NC * _NS
_PPW = _P // _NW
_PCH = _PPW // 2
_TPW = _T // _NW


def _worker_id():
    return lax.axis_index("s") * _NC + lax.axis_index("c")


@functools.lru_cache(maxsize=None)
def _sc_kernels():
    mesh = plsc.VectorSubcoreMesh(
        core_axis_name="c", subcore_axis_name="s",
        num_cores=_NC, num_subcores=_NS)

    @functools.partial(
        pl.kernel,
        out_type=jax.ShapeDtypeStruct((_S, _D), jnp.float32),
        mesh=mesh,
        scratch_types=[
            pltpu.VMEM((_PCH,), jnp.int32),
            pltpu.VMEM((_PCH,), jnp.int32),
            pltpu.VMEM((_PCH,), jnp.int32),
            pltpu.VMEM((_PCH,), jnp.int32),
            pltpu.VMEM((_PCH, _D), jnp.float32),
            pltpu.VMEM((_PCH, _D), jnp.float32),
            pltpu.SemaphoreType.DMA,
            pltpu.SemaphoreType.DMA,
            pltpu.SemaphoreType.DMA,
            pltpu.SemaphoreType.DMA,
        ],
    )
    def sc_dispatch(x_hbm, slots_hbm, xs_hbm,
                    sidx_a, sidx_b, tok_a, tok_b, buf_a, buf_b,
                    sem0, sem1, sem2, sem3):
        wid = _worker_id()
        tok0 = wid * (_PPW // _K)
        pltpu.sync_copy(slots_hbm.at[wid, 0], sidx_a)
        pltpu.sync_copy(slots_hbm.at[wid, 1], sidx_b)
        for k in range(_PCH // 16):
            i = lax.iota(jnp.int32, 16) + (k * 16)
            tok_a[pl.ds(k * 16, 16)] = tok0 + lax.shift_right_logical(i, 1)
            tok_b[pl.ds(k * 16, 16)] = (
                tok0 + lax.shift_right_logical(_PCH + i, 1))
        cp_a = pltpu.async_copy(x_hbm.at[tok_a], buf_a, sem0)
        cp_b = pltpu.async_copy(x_hbm.at[tok_b], buf_b, sem1)
        cp_a.wait()
        st_a = pltpu.async_copy(buf_a, xs_hbm.at[sidx_a], sem2)
        cp_b.wait()
        st_b = pltpu.async_copy(buf_b, xs_hbm.at[sidx_b], sem3)
        st_a.wait()
        st_b.wait()

    @functools.partial(
        pl.kernel,
        out_type=jax.ShapeDtypeStruct((_T, _D), jnp.float32),
        mesh=mesh,
        scratch_types=[
            pltpu.VMEM((_TPW,), jnp.int32),
            pltpu.VMEM((_TPW,), jnp.int32),
            pltpu.VMEM((_TPW, 16), jnp.float32),
            pltpu.VMEM((_TPW, 16), jnp.float32),
            pltpu.VMEM((_TPW, _D), jnp.float32),
            pltpu.VMEM((_TPW, _D), jnp.float32),
            pltpu.SemaphoreType.DMA,
            pltpu.SemaphoreType.DMA,
        ],
    )
    def sc_combine(y_hbm, inv_hbm, g_hbm, out_hbm,
                   idx0, idx1, g0, g1, buf0, buf1, sem0, sem1):
        wid = _worker_id()
        base = wid * _TPW
        pltpu.sync_copy(inv_hbm.at[0, wid], idx0)
        pltpu.sync_copy(inv_hbm.at[1, wid], idx1)
        pltpu.sync_copy(g_hbm.at[0, wid], g0)
        pltpu.sync_copy(g_hbm.at[1, wid], g1)
        cp0 = pltpu.async_copy(y_hbm.at[idx0], buf0, sem0)
        cp1 = pltpu.async_copy(y_hbm.at[idx1], buf1, sem1)
        cp0.wait()
        cp1.wait()

        def row_fn(r, carry):
            g0v = g0[r, :]
            g1v = g1[r, :]
            for c in range(_D // 16):
                sl = pl.ds(c * 16, 16)
                buf0[r, sl] = g0v * buf0[r, sl] + g1v * buf1[r, sl]
            return carry

        lax.fori_loop(0, _TPW, row_fn, 0)
        pltpu.sync_copy(buf0, out_hbm.at[pl.ds(base, _TPW)])

    return sc_dispatch, sc_combine


def _ffn_body(te_ref, par_ref, fi_ref, pf_ref,
              xs_ref, b1_ref, b2_ref, w1_any, w2_any, y_ref,
              wbuf1, wbuf2, semw):
    i = pl.program_id(0)
    s = par_ref[i]

    @pl.when(i == 0)
    def _prolog():
        te = te_ref[0]
        pltpu.make_async_copy(w1_any.at[te], wbuf1.at[0], semw).start()
        pltpu.make_async_copy(w2_any.at[te], wbuf2.at[0], semw).start()

    @pl.when(fi_ref[i] == 1)
    def _group_start():
        te = te_ref[i]
        pltpu.make_async_copy(w1_any.at[te], wbuf1.at[s], semw).wait()
        pltpu.make_async_copy(w2_any.at[te], wbuf2.at[s], semw).wait()

        @pl.when(pf_ref[i] >= 0)
        def _prefetch_next():
            nslot = 1 - s
            pe = pf_ref[i]
            pltpu.make_async_copy(w1_any.at[pe], wbuf1.at[nslot], semw).start()
            pltpu.make_async_copy(w2_any.at[pe], wbuf2.at[nslot], semw).start()

    xg = xs_ref[...]
    h = jnp.dot(xg, wbuf1[s], preferred_element_type=jnp.float32)
    h = jax.nn.gelu(h + b1_ref[0])
    y = jnp.dot(h, wbuf2[s], preferred_element_type=jnp.float32)
    y_ref[...] = y + b2_ref[0]


def _ffn_grid_spec():
    return pltpu.PrefetchScalarGridSpec(
        num_scalar_prefetch=4,
        grid=(_MT,),
        in_specs=[
            pl.BlockSpec((_TILE, _D), lambda i, te, par, fi, pf: (i, 0)),
            pl.BlockSpec((1, 1, _F), lambda i, te, par, fi, pf: (te[i], 0, 0)),
            pl.BlockSpec((1, 1, _D), lambda i, te, par, fi, pf: (te[i], 0, 0)),
            pl.BlockSpec(memory_space=pl.ANY),
            pl.BlockSpec(memory_space=pl.ANY),
        ],
        out_specs=pl.BlockSpec((_TILE, _D), lambda i, te, par, fi, pf: (i, 0)),
        scratch_shapes=[
            pltpu.VMEM((2, _D, _F), jnp.float32),
            pltpu.VMEM((2, _F, _D), jnp.float32),
            pltpu.SemaphoreType.DMA,
        ],
    )


def _ffn(tile_e, parity, first, pf, xs, w1, b1, w2, b2):
    return pl.pallas_call(
        _ffn_body,
        grid_spec=_ffn_grid_spec(),
        out_shape=jax.ShapeDtypeStruct((_S, _D), jnp.float32),
        compiler_params=pltpu.CompilerParams(
            dimension_semantics=("arbitrary",),
        ),
    )(tile_e, parity, first, pf,
      xs, b1[:, None, :], b2[:, None, :], w1, w2)


def _route(x, router_w):
    logits = x @ router_w
    ar = jnp.arange(_E, dtype=jnp.int32)
    i0 = jnp.argmax(logits, axis=-1).astype(jnp.int32)
    v0 = jnp.max(logits, axis=-1)
    masked = jnp.where(ar[None, :] == i0[:, None], -jnp.inf, logits)
    i1 = jnp.argmax(masked, axis=-1).astype(jnp.int32)
    v1 = jnp.max(masked, axis=-1)
    topi = jnp.stack([i0, i1], axis=-1)
    topv = jnp.stack([v0, v1], axis=-1)
    gates = jax.nn.softmax(topv, axis=-1)
    eflat = topi.reshape(-1).astype(jnp.int32)

    onehot = (eflat[:, None] == jnp.arange(_E, dtype=jnp.int32)[None, :])
    csum = jnp.cumsum(onehot.astype(jnp.int32), axis=0)
    counts = csum[-1]
    rank = jnp.take_along_axis(csum, eflat[:, None], axis=1)[:, 0] - 1
    tiles_e = (counts + _TILE - 1) // _TILE
    tile_start = jnp.concatenate(
        [jnp.zeros(1, jnp.int32), jnp.cumsum(tiles_e).astype(jnp.int32)])
    pad_off = tile_start * _TILE
    slot = pad_off[eflat] + rank

    tile_e = jnp.minimum(
        jnp.sum(jnp.arange(_MT, dtype=jnp.int32)[:, None] >= tile_start[None, 1:],
                axis=1),
        _E - 1).astype(jnp.int32)

    first = jnp.concatenate(
        [jnp.ones(1, jnp.int32),
         (tile_e[1:] != tile_e[:-1]).astype(jnp.int32)])
    gid = jnp.cumsum(first) - 1
    parity = (gid % 2).astype(jnp.int32)
    pos = jnp.where(first == 1, jnp.arange(_MT, dtype=jnp.int32), _MT)
    ns = jnp.flip(jax.lax.cummin(jnp.flip(pos)))
    ns = jnp.concatenate([ns[1:], jnp.full(1, _MT, jnp.int32)])
    pf = jnp.where((first == 1) & (ns < _MT),
                   tile_e[jnp.minimum(ns, _MT - 1)], -1).astype(jnp.int32)

    sched = (tile_e, parity, first, pf)
    slots_arr = slot.reshape(_NW, _K, _PCH)
    inv = slot.reshape(_T, _K)
    inv_arr = jnp.stack([inv[:, 0].reshape(_NW, _TPW),
                         inv[:, 1].reshape(_NW, _TPW)])
    g_arr = jnp.broadcast_to(
        jnp.stack([gates[:, 0].reshape(_NW, _TPW),
                   gates[:, 1].reshape(_NW, _TPW)])[..., None],
        (2, _NW, _TPW, 16)).astype(jnp.float32)
    return slots_arr, sched, inv_arr, g_arr


def kernel(x, router_w, w1, b1, w2, b2):
    slots_arr, sched, inv_arr, g_arr = _route(x, router_w)
    sc_dispatch, sc_combine = _sc_kernels()
    xs = sc_dispatch(x, slots_arr)
    y_sorted = _ffn(*sched, xs, w1, b1, w2, b2)
    return sc_combine(y_sorted, inv_arr, g_arr)


# speedup vs baseline: 1.6701x; 1.0194x over previous
import functools

import jax
import jax.numpy as jnp
from jax import lax
from jax.experimental import pallas as pl
from jax.experimental.pallas import tpu as pltpu
from jax.experimental.pallas import tpu_sc as plsc

_E = 8
_K = 2
_T = 2048
_D = 768
_F = 2048
_TILE = 256
_P = _T * _K
_MT = _P // _TILE + _E
_S = _MT * _TILE

_NC = 2
_NS = 16
_NW = _NC * _NS
_PPW = _P // _NW
_PCH = _PPW // 2
_TPW = _T // _NW


def _worker_id():
    return lax.axis_index("s") * _NC + lax.axis_index("c")


@functools.lru_cache(maxsize=None)
def _sc_kernels():
    mesh = plsc.VectorSubcoreMesh(
        core_axis_name="c", subcore_axis_name="s",
        num_cores=_NC, num_subcores=_NS)

    @functools.partial(
        pl.kernel,
        out_type=jax.ShapeDtypeStruct((_S, _D), jnp.float32),
        mesh=mesh,
        scratch_types=[
            pltpu.VMEM((_PCH,), jnp.int32),
            pltpu.VMEM((_PCH,), jnp.int32),
            pltpu.VMEM((_PCH,), jnp.int32),
            pltpu.VMEM((_PCH,), jnp.int32),
            pltpu.VMEM((_PCH, _D), jnp.float32),
            pltpu.VMEM((_PCH, _D), jnp.float32),
            pltpu.SemaphoreType.DMA,
            pltpu.SemaphoreType.DMA,
            pltpu.SemaphoreType.DMA,
            pltpu.SemaphoreType.DMA,
        ],
    )
    def sc_dispatch(x_hbm, slots_hbm, xs_hbm,
                    sidx_a, sidx_b, tok_a, tok_b, buf_a, buf_b,
                    sem0, sem1, sem2, sem3):
        wid = _worker_id()
        tok0 = wid * (_PPW // _K)
        pltpu.sync_copy(slots_hbm.at[wid, 0], sidx_a)
        pltpu.sync_copy(slots_hbm.at[wid, 1], sidx_b)
        for k in range(_PCH // 16):
            i = lax.iota(jnp.int32, 16) + (k * 16)
            tok_a[pl.ds(k * 16, 16)] = tok0 + lax.shift_right_logical(i, 1)
            tok_b[pl.ds(k * 16, 16)] = (
                tok0 + lax.shift_right_logical(_PCH + i, 1))
        cp_a = pltpu.async_copy(x_hbm.at[tok_a], buf_a, sem0)
        cp_b = pltpu.async_copy(x_hbm.at[tok_b], buf_b, sem1)
        cp_a.wait()
        st_a = pltpu.async_copy(buf_a, xs_hbm.at[sidx_a], sem2)
        cp_b.wait()
        st_b = pltpu.async_copy(buf_b, xs_hbm.at[sidx_b], sem3)
        st_a.wait()
        st_b.wait()

    @functools.partial(
        pl.kernel,
        out_type=jax.ShapeDtypeStruct((_T, _D), jnp.float32),
        mesh=mesh,
        scratch_types=[
            pltpu.VMEM((_TPW,), jnp.int32),
            pltpu.VMEM((_TPW,), jnp.int32),
            pltpu.VMEM((_TPW, 16), jnp.float32),
            pltpu.VMEM((_TPW, 16), jnp.float32),
            pltpu.VMEM((_TPW, _D), jnp.float32),
            pltpu.VMEM((_TPW, _D), jnp.float32),
            pltpu.SemaphoreType.DMA,
            pltpu.SemaphoreType.DMA,
        ],
    )
    def sc_combine(y_hbm, inv_hbm, g_hbm, out_hbm,
                   idx0, idx1, g0, g1, buf0, buf1, sem0, sem1):
        wid = _worker_id()
        base = wid * _TPW
        pltpu.sync_copy(inv_hbm.at[0, wid], idx0)
        pltpu.sync_copy(inv_hbm.at[1, wid], idx1)
        pltpu.sync_copy(g_hbm.at[0, wid], g0)
        pltpu.sync_copy(g_hbm.at[1, wid], g1)
        cp0 = pltpu.async_copy(y_hbm.at[idx0], buf0, sem0)
        cp1 = pltpu.async_copy(y_hbm.at[idx1], buf1, sem1)
        cp0.wait()
        cp1.wait()

        def row_fn(r, carry):
            g0v = g0[r, :]
            g1v = g1[r, :]
            for c in range(_D // 16):
                sl = pl.ds(c * 16, 16)
                buf0[r, sl] = g0v * buf0[r, sl] + g1v * buf1[r, sl]
            return carry

        lax.fori_loop(0, _TPW, row_fn, 0)
        pltpu.sync_copy(buf0, out_hbm.at[pl.ds(base, _TPW)])

    return sc_dispatch, sc_combine


def _ffn_body(te_ref, par_ref, fi_ref, pf_ref,
              xs_ref, b1_ref, b2_ref, w1_any, w2_any, y_ref,
              wbuf1, wbuf2, semw):
    i = pl.program_id(0)
    s = par_ref[i]

    @pl.when(i == 0)
    def _prolog():
        te = te_ref[0]
        pltpu.make_async_copy(w1_any.at[te], wbuf1.at[0], semw).start()
        pltpu.make_async_copy(w2_any.at[te], wbuf2.at[0], semw).start()

    @pl.when(fi_ref[i] == 1)
    def _group_start():
        te = te_ref[i]
        pltpu.make_async_copy(w1_any.at[te], wbuf1.at[s], semw).wait()
        pltpu.make_async_copy(w2_any.at[te], wbuf2.at[s], semw).wait()

        @pl.when(pf_ref[i] >= 0)
        def _prefetch_next():
            nslot = 1 - s
            pe = pf_ref[i]
            pltpu.make_async_copy(w1_any.at[pe], wbuf1.at[nslot], semw).start()
            pltpu.make_async_copy(w2_any.at[pe], wbuf2.at[nslot], semw).start()

    xg = xs_ref[...]
    h = jnp.dot(xg, wbuf1[s], preferred_element_type=jnp.float32)
    h = jax.nn.gelu(h + b1_ref[0])
    y = jnp.dot(h, wbuf2[s], preferred_element_type=jnp.float32)
    y_ref[...] = y + b2_ref[0]


def _ffn_grid_spec():
    return pltpu.PrefetchScalarGridSpec(
        num_scalar_prefetch=4,
        grid=(_MT,),
        in_specs=[
            pl.BlockSpec((_TILE, _D), lambda i, te, par, fi, pf: (i, 0)),
            pl.BlockSpec((1, 1, _F), lambda i, te, par, fi, pf: (te[i], 0, 0)),
            pl.BlockSpec((1, 1, _D), lambda i, te, par, fi, pf: (te[i], 0, 0)),
            pl.BlockSpec(memory_space=pl.ANY),
            pl.BlockSpec(memory_space=pl.ANY),
        ],
        out_specs=pl.BlockSpec((_TILE, _D), lambda i, te, par, fi, pf: (i, 0)),
        scratch_shapes=[
            pltpu.VMEM((2, _D, _F), jnp.float32),
            pltpu.VMEM((2, _F, _D), jnp.float32),
            pltpu.SemaphoreType.DMA,
        ],
    )


def _ffn(tile_e, parity, first, pf, xs, w1, b1, w2, b2):
    return pl.pallas_call(
        _ffn_body,
        grid_spec=_ffn_grid_spec(),
        out_shape=jax.ShapeDtypeStruct((_S, _D), jnp.float32),
        compiler_params=pltpu.CompilerParams(
            dimension_semantics=("arbitrary",),
        ),
    )(tile_e, parity, first, pf,
      xs, b1[:, None, :], b2[:, None, :], w1, w2)


def _route(x, router_w):
    logits = x @ router_w
    ar = jnp.arange(_E, dtype=jnp.int32)
    i0 = jnp.argmax(logits, axis=-1).astype(jnp.int32)
    v0 = jnp.max(logits, axis=-1)
    masked = jnp.where(ar[None, :] == i0[:, None], -jnp.inf, logits)
    i1 = jnp.argmax(masked, axis=-1).astype(jnp.int32)
    v1 = jnp.max(masked, axis=-1)
    topi = jnp.stack([i0, i1], axis=-1)
    topv = jnp.stack([v0, v1], axis=-1)
    gates = jax.nn.softmax(topv, axis=-1)
    eflat = topi.reshape(-1).astype(jnp.int32)

    onehot = (eflat[:, None] == ar[None, :]).astype(jnp.float32)
    _B = 128
    _NB = _P // _B
    oh = onehot.reshape(_NB, _B, _E)
    tril_b = (jnp.arange(_B)[:, None] >= jnp.arange(_B)[None, :]).astype(jnp.float32)
    intra = jnp.einsum('rc,bce->bre', tril_b, oh,
                       precision=lax.Precision.DEFAULT)
    btot = jnp.sum(oh, axis=1)
    tril_nb = (jnp.arange(_NB)[:, None] > jnp.arange(_NB)[None, :]).astype(jnp.float32)
    bpre = jnp.dot(tril_nb, btot, precision=lax.Precision.DEFAULT)
    csum = (intra + bpre[:, None, :]).reshape(_P, _E)
    counts = (btot.sum(axis=0)).astype(jnp.int32)
    rank = jnp.take_along_axis(csum, eflat[:, None], axis=1)[:, 0].astype(jnp.int32) - 1
    tiles_e = (counts + _TILE - 1) // _TILE
    ar1 = jnp.arange(_E + 1, dtype=jnp.int32)
    tile_start = jnp.sum(
        jnp.where(ar1[:, None] > ar[None, :], tiles_e[None, :], 0),
        axis=1).astype(jnp.int32)
    pad_off = tile_start * _TILE
    slot = pad_off[eflat] + rank

    tile_e = jnp.minimum(
        jnp.sum(jnp.arange(_MT, dtype=jnp.int32)[:, None] >= tile_start[None, 1:],
                axis=1),
        _E - 1).astype(jnp.int32)

    first = jnp.concatenate(
        [jnp.ones(1, jnp.int32),
         (tile_e[1:] != tile_e[:-1]).astype(jnp.int32)])
    arm = jnp.arange(_MT, dtype=jnp.int32)
    gid = jnp.sum(jnp.where(arm[:, None] >= arm[None, :],
                            first[None, :], 0), axis=1) - 1
    parity = (gid % 2).astype(jnp.int32)
    pos = jnp.where(first == 1, arm, _MT)
    ns = jnp.min(jnp.where(arm[None, :] > arm[:, None],
                           pos[None, :], _MT), axis=1)
    pf = jnp.where((first == 1) & (ns < _MT),
                   tile_e[jnp.minimum(ns, _MT - 1)], -1).astype(jnp.int32)

    sched = (tile_e, parity, first, pf)
    slots_arr = slot.reshape(_NW, _K, _PCH)
    inv = slot.reshape(_T, _K)
    inv_arr = jnp.stack([inv[:, 0].reshape(_NW, _TPW),
                         inv[:, 1].reshape(_NW, _TPW)])
    g_arr = jnp.broadcast_to(
        jnp.stack([gates[:, 0].reshape(_NW, _TPW),
                   gates[:, 1].reshape(_NW, _TPW)])[..., None],
        (2, _NW, _TPW, 16)).astype(jnp.float32)
    return slots_arr, sched, inv_arr, g_arr


def kernel(x, router_w, w1, b1, w2, b2):
    slots_arr, sched, inv_arr, g_arr = _route(x, router_w)
    sc_dispatch, sc_combine = _sc_kernels()
    xs = sc_dispatch(x, slots_arr)
    y_sorted = _ffn(*sched, xs, w1, b1, w2, b2)
    return sc_combine(y_sorted, inv_arr, g_arr)


# speedup vs baseline: 1.8688x; 1.1190x over previous
import functools

import jax
import jax.numpy as jnp
from jax import lax
from jax.experimental import pallas as pl
from jax.experimental.pallas import tpu as pltpu
from jax.experimental.pallas import tpu_sc as plsc

_E = 8
_K = 2
_T = 2048
_D = 768
_F = 2048
_TILE = 256
_P = _T * _K
_MT = _P // _TILE + _E
_S = _MT * _TILE

_NC = 2
_NS = 16
_NW = _NC * _NS
_PPW = _P // _NW
_PCH = _PPW // 2
_TPW = _T // _NW


def _worker_id():
    return lax.axis_index("s") * _NC + lax.axis_index("c")


@functools.lru_cache(maxsize=None)
def _sc_kernels():
    mesh = plsc.VectorSubcoreMesh(
        core_axis_name="c", subcore_axis_name="s",
        num_cores=_NC, num_subcores=_NS)

    @functools.partial(
        pl.kernel,
        out_type=jax.ShapeDtypeStruct((_S, _D), jnp.float32),
        mesh=mesh,
        scratch_types=[
            pltpu.VMEM((_PCH,), jnp.int32),
            pltpu.VMEM((_PCH,), jnp.int32),
            pltpu.VMEM((_PCH,), jnp.int32),
            pltpu.VMEM((_PCH,), jnp.int32),
            pltpu.VMEM((_PCH, _D), jnp.float32),
            pltpu.VMEM((_PCH, _D), jnp.float32),
            pltpu.SemaphoreType.DMA,
            pltpu.SemaphoreType.DMA,
            pltpu.SemaphoreType.DMA,
            pltpu.SemaphoreType.DMA,
        ],
    )
    def sc_dispatch(x_hbm, slots_hbm, xs_hbm,
                    sidx_a, sidx_b, tok_a, tok_b, buf_a, buf_b,
                    sem0, sem1, sem2, sem3):
        wid = _worker_id()
        tok0 = wid * (_PPW // _K)
        pltpu.sync_copy(slots_hbm.at[wid, 0], sidx_a)
        pltpu.sync_copy(slots_hbm.at[wid, 1], sidx_b)
        for k in range(_PCH // 16):
            i = lax.iota(jnp.int32, 16) + (k * 16)
            tok_a[pl.ds(k * 16, 16)] = tok0 + lax.shift_right_logical(i, 1)
            tok_b[pl.ds(k * 16, 16)] = (
                tok0 + lax.shift_right_logical(_PCH + i, 1))
        cp_a = pltpu.async_copy(x_hbm.at[tok_a], buf_a, sem0)
        cp_b = pltpu.async_copy(x_hbm.at[tok_b], buf_b, sem1)
        cp_a.wait()
        st_a = pltpu.async_copy(buf_a, xs_hbm.at[sidx_a], sem2)
        cp_b.wait()
        st_b = pltpu.async_copy(buf_b, xs_hbm.at[sidx_b], sem3)
        st_a.wait()
        st_b.wait()

    @functools.partial(
        pl.kernel,
        out_type=jax.ShapeDtypeStruct((_T, _D), jnp.float32),
        mesh=mesh,
        scratch_types=[
            pltpu.VMEM((_TPW,), jnp.int32),
            pltpu.VMEM((_TPW,), jnp.int32),
            pltpu.VMEM((_TPW, 16), jnp.float32),
            pltpu.VMEM((_TPW, 16), jnp.float32),
            pltpu.VMEM((_TPW, _D), jnp.float32),
            pltpu.VMEM((_TPW, _D), jnp.float32),
            pltpu.SemaphoreType.DMA,
            pltpu.SemaphoreType.DMA,
        ],
    )
    def sc_combine(y_hbm, inv_hbm, g_hbm, out_hbm,
                   idx0, idx1, g0, g1, buf0, buf1, sem0, sem1):
        wid = _worker_id()
        base = wid * _TPW
        pltpu.sync_copy(inv_hbm.at[0, wid], idx0)
        pltpu.sync_copy(inv_hbm.at[1, wid], idx1)
        pltpu.sync_copy(g_hbm.at[0, wid], g0)
        pltpu.sync_copy(g_hbm.at[1, wid], g1)
        cp0 = pltpu.async_copy(y_hbm.at[idx0], buf0, sem0)
        cp1 = pltpu.async_copy(y_hbm.at[idx1], buf1, sem1)
        cp0.wait()
        cp1.wait()

        def row_fn(r, carry):
            g0v = g0[r, :]
            g1v = g1[r, :]
            for c in range(_D // 16):
                sl = pl.ds(c * 16, 16)
                buf0[r, sl] = g0v * buf0[r, sl] + g1v * buf1[r, sl]
            return carry

        lax.fori_loop(0, _TPW, row_fn, 0)
        pltpu.sync_copy(buf0, out_hbm.at[pl.ds(base, _TPW)])

    return sc_dispatch, sc_combine


def _ffn_body(te_ref, par_ref, fi_ref, pf_ref,
              xs_ref, b1_ref, b2_ref, w1_any, w2_any, y_ref,
              wbuf1, wbuf2, semw):
    i = pl.program_id(0)
    s = par_ref[i]

    @pl.when(i == 0)
    def _prolog():
        te = te_ref[0]
        pltpu.make_async_copy(w1_any.at[te], wbuf1.at[0], semw).start()
        pltpu.make_async_copy(w2_any.at[te], wbuf2.at[0], semw).start()

    @pl.when(fi_ref[i] == 1)
    def _group_start():
        te = te_ref[i]
        pltpu.make_async_copy(w1_any.at[te], wbuf1.at[s], semw).wait()
        pltpu.make_async_copy(w2_any.at[te], wbuf2.at[s], semw).wait()

        @pl.when(pf_ref[i] >= 0)
        def _prefetch_next():
            nslot = 1 - s
            pe = pf_ref[i]
            pltpu.make_async_copy(w1_any.at[pe], wbuf1.at[nslot], semw).start()
            pltpu.make_async_copy(w2_any.at[pe], wbuf2.at[nslot], semw).start()

    xg = xs_ref[...]
    h = jnp.dot(xg, wbuf1[s], preferred_element_type=jnp.float32)
    h = jax.nn.gelu(h + b1_ref[0])
    y = jnp.dot(h, wbuf2[s], preferred_element_type=jnp.float32)
    y_ref[...] = y + b2_ref[0]


def _ffn_grid_spec():
    return pltpu.PrefetchScalarGridSpec(
        num_scalar_prefetch=4,
        grid=(_MT,),
        in_specs=[
            pl.BlockSpec((_TILE, _D), lambda i, te, par, fi, pf: (i, 0)),
            pl.BlockSpec((1, 1, _F), lambda i, te, par, fi, pf: (te[i], 0, 0)),
            pl.BlockSpec((1, 1, _D), lambda i, te, par, fi, pf: (te[i], 0, 0)),
            pl.BlockSpec(memory_space=pl.ANY),
            pl.BlockSpec(memory_space=pl.ANY),
        ],
        out_specs=pl.BlockSpec((_TILE, _D), lambda i, te, par, fi, pf: (i, 0)),
        scratch_shapes=[
            pltpu.VMEM((2, _D, _F), jnp.float32),
            pltpu.VMEM((2, _F, _D), jnp.float32),
            pltpu.SemaphoreType.DMA,
        ],
    )


def _ffn(tile_e, parity, first, pf, xs, w1, b1, w2, b2):
    return pl.pallas_call(
        _ffn_body,
        grid_spec=_ffn_grid_spec(),
        out_shape=jax.ShapeDtypeStruct((_S, _D), jnp.float32),
        compiler_params=pltpu.CompilerParams(
            dimension_semantics=("arbitrary",),
        ),
    )(tile_e, parity, first, pf,
      xs, b1[:, None, :], b2[:, None, :], w1, w2)


def _route(x, router_w):
    logits = x @ router_w
    ar = jnp.arange(_E, dtype=jnp.int32)
    i0 = jnp.argmax(logits, axis=-1).astype(jnp.int32)
    v0 = jnp.max(logits, axis=-1)
    masked = jnp.where(ar[None, :] == i0[:, None], -jnp.inf, logits)
    i1 = jnp.argmax(masked, axis=-1).astype(jnp.int32)
    v1 = jnp.max(masked, axis=-1)
    topi = jnp.stack([i0, i1], axis=-1)
    topv = jnp.stack([v0, v1], axis=-1)
    gates = jax.nn.softmax(topv, axis=-1)
    eflat = topi.reshape(-1).astype(jnp.int32)

    onehot = (eflat[:, None] == ar[None, :]).astype(jnp.float32)
    _B = 128
    _NB = _P // _B
    oh = onehot.reshape(_NB, _B, _E)
    tril_b = (jnp.arange(_B)[:, None] >= jnp.arange(_B)[None, :]).astype(jnp.float32)
    intra = jnp.einsum('rc,bce->bre', tril_b, oh,
                       precision=lax.Precision.DEFAULT)
    btot = jnp.sum(oh, axis=1)
    tril_nb = (jnp.arange(_NB)[:, None] > jnp.arange(_NB)[None, :]).astype(jnp.float32)
    bpre = jnp.dot(tril_nb, btot, precision=lax.Precision.DEFAULT)
    csum = (intra + bpre[:, None, :]).reshape(_P, _E)
    counts = (btot.sum(axis=0)).astype(jnp.int32)
    rank = jnp.sum(csum * onehot, axis=1).astype(jnp.int32) - 1
    tiles_e = (counts + _TILE - 1) // _TILE
    ar1 = jnp.arange(_E + 1, dtype=jnp.int32)
    tile_start = jnp.sum(
        jnp.where(ar1[:, None] > ar[None, :], tiles_e[None, :], 0),
        axis=1).astype(jnp.int32)
    pad_off = tile_start * _TILE
    pad_of_pair = jnp.sum(onehot * pad_off[:_E].astype(jnp.float32)[None, :],
                          axis=1).astype(jnp.int32)
    slot = pad_of_pair + rank

    tile_e = jnp.minimum(
        jnp.sum(jnp.arange(_MT, dtype=jnp.int32)[:, None] >= tile_start[None, 1:],
                axis=1),
        _E - 1).astype(jnp.int32)

    first = jnp.concatenate(
        [jnp.ones(1, jnp.int32),
         (tile_e[1:] != tile_e[:-1]).astype(jnp.int32)])
    arm = jnp.arange(_MT, dtype=jnp.int32)
    gid = jnp.sum(jnp.where(arm[:, None] >= arm[None, :],
                            first[None, :], 0), axis=1) - 1
    parity = (gid % 2).astype(jnp.int32)
    pos = jnp.where(first == 1, arm, _MT)
    ns = jnp.min(jnp.where(arm[None, :] > arm[:, None],
                           pos[None, :], _MT), axis=1)
    te_at_ns = jnp.sum(
        jnp.where(ns[:, None] == arm[None, :], tile_e[None, :], 0),
        axis=1)
    pf = jnp.where((first == 1) & (ns < _MT),
                   te_at_ns, -1).astype(jnp.int32)

    sched = (tile_e, parity, first, pf)
    slots_arr = slot.reshape(_NW, _K, _PCH)
    inv = slot.reshape(_T, _K)
    inv_arr = jnp.stack([inv[:, 0].reshape(_NW, _TPW),
                         inv[:, 1].reshape(_NW, _TPW)])
    g_arr = jnp.broadcast_to(
        jnp.stack([gates[:, 0].reshape(_NW, _TPW),
                   gates[:, 1].reshape(_NW, _TPW)])[..., None],
        (2, _NW, _TPW, 16)).astype(jnp.float32)
    return slots_arr, sched, inv_arr, g_arr


def kernel(x, router_w, w1, b1, w2, b2):
    slots_arr, sched, inv_arr, g_arr = _route(x, router_w)
    sc_dispatch, sc_combine = _sc_kernels()
    xs = sc_dispatch(x, slots_arr)
    y_sorted = _ffn(*sched, xs, w1, b1, w2, b2)
    return sc_combine(y_sorted, inv_arr, g_arr)
